# Initial kernel scaffold; baseline (speedup 1.0000x reference)
import jax
import jax.numpy as jnp
from jax.experimental import pallas as pl


def kernel(nodes, edges, senders, receivers, node_types, params):
    raise NotImplementedError("write your pallas kernel here")


# speedup vs baseline: 1.4192x; 1.4192x over previous
import functools

import jax
import jax.numpy as jnp
from jax import lax
from jax.experimental import pallas as pl
from jax.experimental.pallas import tpu as pltpu
from jax.experimental.pallas import tpu_sc as plsc

N = 10000
E = 160000
D = 128
DE = 16
MSG = 128
L = 4

_f32 = jnp.float32
_PREC = jax.lax.Precision.HIGHEST
_PREC_X = jax.lax.Precision.HIGHEST
_i32 = jnp.int32

_NC = 2
_NS = 16
_NW = _NC * _NS
_ECH = 128
_NCHK = E // _ECH
_CBASE = _NCHK // _NW
_CREM = _NCHK % _NW
_GRP = _ECH // 16

_NACC = 10240
_DENR = _NACC // 8

_TE = 1280
_TN = 2048


@functools.cache
def _gather_fn():
    mesh = plsc.VectorSubcoreMesh(core_axis_name="c", subcore_axis_name="s")

    @functools.partial(
        pl.kernel,
        mesh=mesh,
        out_type=(jax.ShapeDtypeStruct((E, D), _f32),
                  jax.ShapeDtypeStruct((E, D), _f32)),
        scratch_types=[
            pltpu.VMEM((_ECH,), _i32),
            pltpu.VMEM((_ECH,), _i32),
            pltpu.VMEM((_ECH, D), _f32),
            pltpu.VMEM((_ECH, D), _f32),
            pltpu.SemaphoreType.DMA,
            pltpu.SemaphoreType.DMA,
        ],
    )
    def gather_k(x_hbm, s_hbm, r_hbm, xs_hbm, xr_hbm,
                 idx_s, idx_r, buf_s, buf_r, sem_s, sem_r):
        wid = lax.axis_index("s") * _NC + lax.axis_index("c")
        nck = jnp.where(wid < _CREM, _CBASE + 1, _CBASE)

        def body(ci, carry):
            off = (wid + _NW * ci) * _ECH
            pltpu.sync_copy(s_hbm.at[pl.ds(off, _ECH)], idx_s)
            pltpu.sync_copy(r_hbm.at[pl.ds(off, _ECH)], idx_r)
            cs = pltpu.async_copy(x_hbm.at[idx_s], buf_s, sem_s)
            cr = pltpu.async_copy(x_hbm.at[idx_r], buf_r, sem_r)
            cs.wait()
            cr.wait()
            pltpu.sync_copy(buf_s, xs_hbm.at[pl.ds(off, _ECH)])
            pltpu.sync_copy(buf_r, xr_hbm.at[pl.ds(off, _ECH)])
            return carry

        lax.fori_loop(0, nck, body, 0)

    return gather_k


@functools.cache
def _scatter_fn():
    mesh = plsc.VectorSubcoreMesh(core_axis_name="c", subcore_axis_name="s")
    rows_n = _NACC // _NS
    rows_d = _DENR // _NS

    @functools.partial(
        pl.kernel,
        mesh=mesh,
        out_type=(jax.ShapeDtypeStruct((_NC, _NACC, MSG), _f32),
                  jax.ShapeDtypeStruct((_NC, _DENR, MSG), _f32)),
        scratch_types=[
            pltpu.VMEM((_ECH,), _i32),
            pltpu.VMEM((_ECH,), _i32),
            pltpu.VMEM((_ECH, MSG), _f32),
            pltpu.VMEM((_ECH, MSG), _f32),
            pltpu.VMEM_SHARED((_NACC, MSG), _f32),
            pltpu.VMEM_SHARED((_DENR, MSG), _f32),
        ],
    )
    def scatter_k(pe_hbm, eb_hbm, r_hbm, z_hbm, num_hbm, den_hbm,
                  idx_r, idx_d, buf, bufd, accn, accd):
        cid = lax.axis_index("c")
        sid = lax.axis_index("s")
        wid = sid * _NC + cid
        pltpu.sync_copy(z_hbm, accn.at[pl.ds(sid * rows_n, rows_n)])
        pltpu.sync_copy(z_hbm.at[pl.ds(0, rows_d)],
                        accd.at[pl.ds(sid * rows_d, rows_d)])
        plsc.subcore_barrier()
        nck = jnp.where(wid < _CREM, _CBASE + 1, _CBASE)

        def body(ci, carry):
            off = (wid + _NW * ci) * _ECH
            pltpu.sync_copy(r_hbm.at[pl.ds(off, _ECH)], idx_r)
            pltpu.sync_copy(pe_hbm.at[pl.ds(off, _ECH)], buf)
            pltpu.sync_copy(eb_hbm.at[pl.ds(off, _ECH)], bufd)
            for g in range(_GRP):
                rv = idx_r[pl.ds(g * 16, 16)]
                idx_d[pl.ds(g * 16, 16)] = lax.shift_right_logical(rv, 3)
            pltpu.sync_copy(buf, accn.at[idx_r], add=True)
            pltpu.sync_copy(bufd, accd.at[idx_d], add=True)
            return carry

        lax.fori_loop(0, nck, body, 0)
        plsc.subcore_barrier()
        pltpu.sync_copy(accn.at[pl.ds(sid * rows_n, rows_n)],
                        num_hbm.at[cid, pl.ds(sid * rows_n, rows_n)])
        pltpu.sync_copy(accd.at[pl.ds(sid * rows_d, rows_d)],
                        den_hbm.at[cid, pl.ds(sid * rows_d, rows_d)])

    return scatter_k


def _swish(x):
    return x * jax.nn.sigmoid(x)


def _edge_body(ed_ref, xs_ref, xr_ref, rcv_ref, w1, b1, w2, b2,
               w3, b3, wg1, bg1, wg2, bg2, watt, pe_ref, eb_ref):
    feats = jnp.concatenate([ed_ref[...], xs_ref[...], xr_ref[...]], axis=1)
    h = jnp.dot(feats, w1[...], preferred_element_type=_f32, precision=_PREC)
    h = _swish(h + b1[...])
    h = _swish(jnp.dot(h, w2[...], preferred_element_type=_f32, precision=_PREC) + b2[...])
    msgs = jnp.dot(h, w3[...], preferred_element_type=_f32, precision=_PREC) + b3[...]
    g = _swish(jnp.dot(msgs, wg1[...], preferred_element_type=_f32, precision=_PREC) + bg1[...])
    g = jnp.dot(g, wg2[...], preferred_element_type=_f32, precision=_PREC) + bg2[...]
    s = jnp.dot(g, watt[...], preferred_element_type=_f32, precision=_PREC)
    e = jnp.exp(s)
    pe_ref[...] = e * msgs
    te = pe_ref.shape[0]
    r = rcv_ref[...]
    j16 = lax.broadcasted_iota(_i32, (te, 1), 0) & 15
    lane = ((r & 7) << 4) + j16
    l1 = lax.broadcasted_iota(_i32, (te, MSG), 1)
    eb_ref[...] = e * (l1 == lane).astype(_f32)


def _full(shape):
    return pl.BlockSpec(shape, lambda *_: tuple(0 for _ in shape))


def _edge_call(ed, xs, xr, rcv3, p):
    w1 = p["m1"][0]
    watt = p["att"][0]
    ne = xs.shape[0]
    grid = (ne // _TE,)
    return pl.pallas_call(
        _edge_body,
        grid=grid,
        in_specs=[
            pl.BlockSpec((_TE, DE), lambda i: (i, 0)),
            pl.BlockSpec((_TE, D), lambda i: (i, 0)),
            pl.BlockSpec((_TE, D), lambda i: (i, 0)),
            pl.BlockSpec((_TE, 1), lambda i: (i, 0)),
            _full(w1.shape),
            _full(p["m1"][1].shape),
            _full(p["m2"][0].shape), _full(p["m2"][1].shape),
            _full(p["m3"][0].shape), _full(p["m3"][1].shape),
            _full(p["g1"][0].shape), _full(p["g1"][1].shape),
            _full(p["g2"][0].shape), _full(p["g2"][1].shape),
            _full(watt.shape),
        ],
        out_specs=(pl.BlockSpec((_TE, MSG), lambda i: (i, 0)),
                   pl.BlockSpec((_TE, MSG), lambda i: (i, 0))),
        out_shape=(jax.ShapeDtypeStruct((ne, MSG), _f32),
                   jax.ShapeDtypeStruct((ne, MSG), _f32)),
        compiler_params=pltpu.CompilerParams(
            dimension_semantics=("arbitrary",)),
    )(ed, xs, xr, rcv3, w1, p["m1"][1], p["m2"][0], p["m2"][1],
      p["m3"][0], p["m3"][1], p["g1"][0], p["g1"][1], p["g2"][0], p["g2"][1],
      watt)


def _node_body(x_ref, pn0_ref, pn1_ref, pd0_ref, pd1_ref,
               w1, b1, w2, b2, w3, b3, out_ref):
    num = pn0_ref[0] + pn1_ref[0]
    d3 = pd0_ref[0] + pd1_ref[0]
    i0 = lax.broadcasted_iota(_i32, (_TN, _TN // 8), 0)
    s0 = lax.broadcasted_iota(_i32, (_TN, _TN // 8), 1)
    rsel = (lax.shift_right_logical(i0, 3) == s0).astype(_f32)
    t = jnp.dot(rsel, d3, preferred_element_type=_f32, precision=_PREC_X)
    i1 = lax.broadcasted_iota(_i32, (_TN, MSG), 0)
    l1 = lax.broadcasted_iota(_i32, (_TN, MSG), 1)
    msk = ((i1 & 7) == lax.shift_right_logical(l1, 4)).astype(_f32)
    den = jnp.sum(t * msk, axis=1, keepdims=True)
    aggr = jnp.where(den > 0.0, num / den, 0.0)
    x = x_ref[...]
    u = jnp.concatenate([x, aggr], axis=1)
    u = jnp.dot(u, w1[...], preferred_element_type=_f32, precision=_PREC)
    u = _swish(u + b1[...])
    u = _swish(jnp.dot(u, w2[...], preferred_element_type=_f32, precision=_PREC) + b2[...])
    upd = jnp.dot(u, w3[...], preferred_element_type=_f32, precision=_PREC) + b3[...]
    out_ref[...] = upd + 0.5 * x


def _node_call(x, pnum, pden, p):
    w1 = p["u1"][0]
    grid = (pl.cdiv(N, _TN),)
    return pl.pallas_call(
        _node_body,
        grid=grid,
        in_specs=[
            pl.BlockSpec((_TN, D), lambda i: (i, 0)),
            pl.BlockSpec((1, _TN, MSG), lambda i: (0, i, 0)),
            pl.BlockSpec((1, _TN, MSG), lambda i: (1, i, 0)),
            pl.BlockSpec((1, _TN // 8, MSG), lambda i: (0, i, 0)),
            pl.BlockSpec((1, _TN // 8, MSG), lambda i: (1, i, 0)),
            _full(w1.shape), _full(p["u1"][1].shape),
            _full(p["u2"][0].shape), _full(p["u2"][1].shape),
            _full(p["u3"][0].shape), _full(p["u3"][1].shape),
        ],
        out_specs=pl.BlockSpec((_TN, D), lambda i: (i, 0)),
        out_shape=jax.ShapeDtypeStruct((N, D), _f32),
        compiler_params=pltpu.CompilerParams(
            dimension_semantics=("arbitrary",)),
    )(x, pnum, pnum, pden, pden, w1, p["u1"][1], p["u2"][0],
      p["u2"][1], p["u3"][0], p["u3"][1])


def _head_body(x_ref, w1, b1, w2, b2, w3, b3, wo, bo, out_ref):
    h = _swish(jnp.dot(x_ref[...], w1[...], preferred_element_type=_f32, precision=_PREC)
               + b1[...])
    h = _swish(jnp.dot(h, w2[...], preferred_element_type=_f32, precision=_PREC) + b2[...])
    h = _swish(jnp.dot(h, w3[...], preferred_element_type=_f32, precision=_PREC) + b3[...])
    s = jnp.dot(h, wo[...], preferred_element_type=_f32, precision=_PREC) + bo[0]
    out_ref[...] = jnp.tanh(s)


def _head_call(x8, params):
    wo = params["hout"][0]
    return pl.pallas_call(
        _head_body,
        in_specs=[
            pl.BlockSpec(x8.shape, lambda: (0, 0)),
            _full(params["h1"][0].shape), _full(params["h1"][1].shape),
            _full(params["h2"][0].shape), _full(params["h2"][1].shape),
            _full(params["h3"][0].shape), _full(params["h3"][1].shape),
            _full(wo.shape),
            pl.BlockSpec(memory_space=pltpu.SMEM),
        ],
        out_specs=pl.BlockSpec((8, 1), lambda: (0, 0)),
        out_shape=jax.ShapeDtypeStruct((8, 1), _f32),
    )(x8, params["h1"][0], params["h1"][1], params["h2"][0], params["h2"][1],
      params["h3"][0], params["h3"][1], wo, params["hout"][1])


def kernel(nodes, edges, senders, receivers, node_types, params):
    del node_types
    gather = _gather_fn()
    scatter = _scatter_fn()
    zrows = jnp.zeros((_NACC // _NS, MSG), _f32)
    rcv2 = receivers.reshape(E, 1)
    x = nodes
    for p in params["layers"]:
        xs, xr = gather(x, senders, receivers)
        pe, eb = _edge_call(edges, xs, xr, rcv2, p)
        pnum, pden = scatter(pe, eb, receivers, zrows)
        x = _node_call(x, pnum, pden, p)
    cbf = _head_call(x[0:8], params)
    return cbf[0, 0]


# speedup vs baseline: 2.3695x; 1.6696x over previous
import functools

import jax
import jax.numpy as jnp
from jax import lax
from jax.experimental import pallas as pl
from jax.experimental.pallas import tpu as pltpu
from jax.experimental.pallas import tpu_sc as plsc

N = 10000
E = 160000
D = 128
DE = 16
MSG = 128
L = 4

_f32 = jnp.float32
_PREC = jax.lax.Precision.HIGHEST
_PREC_X = jax.lax.Precision.HIGHEST
_i32 = jnp.int32

_NC = 2
_NS = 16
_NW = _NC * _NS
_ECH = 128
_NCHK = E // _ECH
_CBASE = _NCHK // _NW
_CREM = _NCHK % _NW
_GRP = _ECH // 16

_NACC = 10240
_DENR = _NACC // 8

_TE = 1280
_TN = 2048


@functools.cache
def _gather_fn():
    mesh = plsc.VectorSubcoreMesh(core_axis_name="c", subcore_axis_name="s")

    @functools.partial(
        pl.kernel,
        mesh=mesh,
        out_type=(jax.ShapeDtypeStruct((E, D), _f32),
                  jax.ShapeDtypeStruct((E, D), _f32)),
        scratch_types=[
            pltpu.VMEM((_ECH,), _i32),
            pltpu.VMEM((_ECH,), _i32),
            pltpu.VMEM((_ECH, D), _f32),
            pltpu.VMEM((_ECH, D), _f32),
            pltpu.SemaphoreType.DMA,
            pltpu.SemaphoreType.DMA,
        ],
    )
    def gather_k(x_hbm, s_hbm, r_hbm, xs_hbm, xr_hbm,
                 idx_s, idx_r, buf_s, buf_r, sem_s, sem_r):
        wid = lax.axis_index("s") * _NC + lax.axis_index("c")
        nck = jnp.where(wid < _CREM, _CBASE + 1, _CBASE)

        def body(ci, carry):
            off = (wid + _NW * ci) * _ECH
            pltpu.sync_copy(s_hbm.at[pl.ds(off, _ECH)], idx_s)
            pltpu.sync_copy(r_hbm.at[pl.ds(off, _ECH)], idx_r)
            cs = pltpu.async_copy(x_hbm.at[idx_s], buf_s, sem_s)
            cr = pltpu.async_copy(x_hbm.at[idx_r], buf_r, sem_r)
            cs.wait()
            cr.wait()
            pltpu.sync_copy(buf_s, xs_hbm.at[pl.ds(off, _ECH)])
            pltpu.sync_copy(buf_r, xr_hbm.at[pl.ds(off, _ECH)])
            return carry

        lax.fori_loop(0, nck, body, 0)

    return gather_k


@functools.cache
def _scatter_fn():
    mesh = plsc.VectorSubcoreMesh(core_axis_name="c", subcore_axis_name="s")
    rows_n = _NACC // _NS
    rows_d = _DENR // _NS

    @functools.partial(
        pl.kernel,
        mesh=mesh,
        out_type=(jax.ShapeDtypeStruct((_NC, _NACC, MSG), _f32),
                  jax.ShapeDtypeStruct((_NC, _DENR, MSG), _f32)),
        scratch_types=[
            pltpu.VMEM((_ECH,), _i32),
            pltpu.VMEM((_ECH,), _i32),
            pltpu.VMEM((_ECH, MSG), _f32),
            pltpu.VMEM((_ECH, MSG), _f32),
            pltpu.VMEM_SHARED((_NACC, MSG), _f32),
            pltpu.VMEM_SHARED((_DENR, MSG), _f32),
        ],
    )
    def scatter_k(pe_hbm, eb_hbm, r_hbm, z_hbm, num_hbm, den_hbm,
                  idx_r, idx_d, buf, bufd, accn, accd):
        cid = lax.axis_index("c")
        sid = lax.axis_index("s")
        wid = sid * _NC + cid
        pltpu.sync_copy(z_hbm, accn.at[pl.ds(sid * rows_n, rows_n)])
        pltpu.sync_copy(z_hbm.at[pl.ds(0, rows_d)],
                        accd.at[pl.ds(sid * rows_d, rows_d)])
        plsc.subcore_barrier()
        nck = jnp.where(wid < _CREM, _CBASE + 1, _CBASE)

        def body(ci, carry):
            off = (wid + _NW * ci) * _ECH
            pltpu.sync_copy(r_hbm.at[pl.ds(off, _ECH)], idx_r)
            pltpu.sync_copy(pe_hbm.at[pl.ds(off, _ECH)], buf)
            pltpu.sync_copy(eb_hbm.at[pl.ds(off, _ECH)], bufd)
            for g in range(_GRP):
                rv = idx_r[pl.ds(g * 16, 16)]
                idx_d[pl.ds(g * 16, 16)] = lax.shift_right_logical(rv, 3)
            pltpu.sync_copy(buf, accn.at[idx_r], add=True)
            pltpu.sync_copy(bufd, accd.at[idx_d], add=True)
            return carry

        lax.fori_loop(0, nck, body, 0)
        plsc.subcore_barrier()
        pltpu.sync_copy(accn.at[pl.ds(sid * rows_n, rows_n)],
                        num_hbm.at[cid, pl.ds(sid * rows_n, rows_n)])
        pltpu.sync_copy(accd.at[pl.ds(sid * rows_d, rows_d)],
                        den_hbm.at[cid, pl.ds(sid * rows_d, rows_d)])

    return scatter_k


def _swish(x):
    return x * jax.nn.sigmoid(x)


_bf16 = jnp.bfloat16


def _split(w):
    hi = w.astype(_bf16)
    return hi, (w - hi.astype(_f32)).astype(_bf16)


def _dot3(a, w_hi, w_lo):
    a_hi = a.astype(_bf16)
    a_lo = (a - a_hi.astype(_f32)).astype(_bf16)
    out = jnp.dot(a_hi, w_lo[...], preferred_element_type=_f32)
    out = out + jnp.dot(a_lo, w_hi[...], preferred_element_type=_f32)
    return out + jnp.dot(a_hi, w_hi[...], preferred_element_type=_f32)


def _edge_body(ed_ref, xs_ref, xr_ref, rcv_ref, w1h, w1l, b1, w2h, w2l, b2,
               w3h, w3l, b3, wg1h, wg1l, bg1, wg2h, wg2l, bg2, wah, wal,
               pe_ref, eb_ref):
    feats = jnp.concatenate([ed_ref[...], xs_ref[...], xr_ref[...]], axis=1)
    h = _swish(_dot3(feats, w1h, w1l) + b1[...])
    h = _swish(_dot3(h, w2h, w2l) + b2[...])
    msgs = _dot3(h, w3h, w3l) + b3[...]
    g = _swish(_dot3(msgs, wg1h, wg1l) + bg1[...])
    g = _dot3(g, wg2h, wg2l) + bg2[...]
    s = _dot3(g, wah, wal)
    e = jnp.exp(s)
    pe_ref[...] = e * msgs
    te = pe_ref.shape[0]
    r = rcv_ref[...]
    j16 = lax.broadcasted_iota(_i32, (te, 1), 0) & 15
    lane = ((r & 7) << 4) + j16
    l1 = lax.broadcasted_iota(_i32, (te, MSG), 1)
    eb_ref[...] = e * (l1 == lane).astype(_f32)


def _full(shape):
    return pl.BlockSpec(shape, lambda *_: tuple(0 for _ in shape))


def _edge_call(ed, xs, xr, rcv3, p):
    w1h, w1l = _split(p["m1"][0])
    w2h, w2l = _split(p["m2"][0])
    w3h, w3l = _split(p["m3"][0])
    wg1h, wg1l = _split(p["g1"][0])
    wg2h, wg2l = _split(p["g2"][0])
    wah, wal = _split(p["att"][0])
    ne = xs.shape[0]
    grid = (ne // _TE,)
    ws = [w1h, w1l, p["m1"][1], w2h, w2l, p["m2"][1], w3h, w3l, p["m3"][1],
          wg1h, wg1l, p["g1"][1], wg2h, wg2l, p["g2"][1], wah, wal]
    return pl.pallas_call(
        _edge_body,
        grid=grid,
        in_specs=[
            pl.BlockSpec((_TE, DE), lambda i: (i, 0)),
            pl.BlockSpec((_TE, D), lambda i: (i, 0)),
            pl.BlockSpec((_TE, D), lambda i: (i, 0)),
            pl.BlockSpec((_TE, 1), lambda i: (i, 0)),
        ] + [_full(w.shape) for w in ws],
        out_specs=(pl.BlockSpec((_TE, MSG), lambda i: (i, 0)),
                   pl.BlockSpec((_TE, MSG), lambda i: (i, 0))),
        out_shape=(jax.ShapeDtypeStruct((ne, MSG), _f32),
                   jax.ShapeDtypeStruct((ne, MSG), _f32)),
        compiler_params=pltpu.CompilerParams(
            dimension_semantics=("arbitrary",)),
    )(ed, xs, xr, rcv3, *ws)


def _node_body(x_ref, pn0_ref, pn1_ref, pd0_ref, pd1_ref,
               w1h, w1l, b1, w2h, w2l, b2, w3h, w3l, b3, out_ref):
    num = pn0_ref[0] + pn1_ref[0]
    d3 = pd0_ref[0] + pd1_ref[0]
    i0 = lax.broadcasted_iota(_i32, (_TN, _TN // 8), 0)
    s0 = lax.broadcasted_iota(_i32, (_TN, _TN // 8), 1)
    rsel = (lax.shift_right_logical(i0, 3) == s0).astype(_bf16)
    d3h = d3.astype(_bf16)
    d3l = (d3 - d3h.astype(_f32)).astype(_bf16)
    t = (jnp.dot(rsel, d3l, preferred_element_type=_f32)
         + jnp.dot(rsel, d3h, preferred_element_type=_f32))
    i1 = lax.broadcasted_iota(_i32, (_TN, MSG), 0)
    l1 = lax.broadcasted_iota(_i32, (_TN, MSG), 1)
    msk = ((i1 & 7) == lax.shift_right_logical(l1, 4)).astype(_f32)
    den = jnp.sum(t * msk, axis=1, keepdims=True)
    aggr = jnp.where(den > 0.0, num / den, 0.0)
    x = x_ref[...]
    u = jnp.concatenate([x, aggr], axis=1)
    u = _swish(_dot3(u, w1h, w1l) + b1[...])
    u = _swish(_dot3(u, w2h, w2l) + b2[...])
    upd = _dot3(u, w3h, w3l) + b3[...]
    out_ref[...] = upd + 0.5 * x


def _node_call(x, pnum, pden, p):
    w1h, w1l = _split(p["u1"][0])
    w2h, w2l = _split(p["u2"][0])
    w3h, w3l = _split(p["u3"][0])
    ws = [w1h, w1l, p["u1"][1], w2h, w2l, p["u2"][1], w3h, w3l, p["u3"][1]]
    grid = (pl.cdiv(N, _TN),)
    return pl.pallas_call(
        _node_body,
        grid=grid,
        in_specs=[
            pl.BlockSpec((_TN, D), lambda i: (i, 0)),
            pl.BlockSpec((1, _TN, MSG), lambda i: (0, i, 0)),
            pl.BlockSpec((1, _TN, MSG), lambda i: (1, i, 0)),
            pl.BlockSpec((1, _TN // 8, MSG), lambda i: (0, i, 0)),
            pl.BlockSpec((1, _TN // 8, MSG), lambda i: (1, i, 0)),
        ] + [_full(w.shape) for w in ws],
        out_specs=pl.BlockSpec((_TN, D), lambda i: (i, 0)),
        out_shape=jax.ShapeDtypeStruct((N, D), _f32),
        compiler_params=pltpu.CompilerParams(
            dimension_semantics=("arbitrary",)),
    )(x, pnum, pnum, pden, pden, *ws)


def _head_body(x_ref, w1, b1, w2, b2, w3, b3, wo, bo, out_ref):
    h = _swish(jnp.dot(x_ref[...], w1[...], preferred_element_type=_f32, precision=_PREC)
               + b1[...])
    h = _swish(jnp.dot(h, w2[...], preferred_element_type=_f32, precision=_PREC) + b2[...])
    h = _swish(jnp.dot(h, w3[...], preferred_element_type=_f32, precision=_PREC) + b3[...])
    s = jnp.dot(h, wo[...], preferred_element_type=_f32, precision=_PREC) + bo[0]
    out_ref[...] = jnp.tanh(s)


def _head_call(x8, params):
    wo = params["hout"][0]
    return pl.pallas_call(
        _head_body,
        in_specs=[
            pl.BlockSpec(x8.shape, lambda: (0, 0)),
            _full(params["h1"][0].shape), _full(params["h1"][1].shape),
            _full(params["h2"][0].shape), _full(params["h2"][1].shape),
            _full(params["h3"][0].shape), _full(params["h3"][1].shape),
            _full(wo.shape),
            pl.BlockSpec(memory_space=pltpu.SMEM),
        ],
        out_specs=pl.BlockSpec((8, 1), lambda: (0, 0)),
        out_shape=jax.ShapeDtypeStruct((8, 1), _f32),
    )(x8, params["h1"][0], params["h1"][1], params["h2"][0], params["h2"][1],
      params["h3"][0], params["h3"][1], wo, params["hout"][1])


def kernel(nodes, edges, senders, receivers, node_types, params):
    del node_types
    gather = _gather_fn()
    scatter = _scatter_fn()
    zrows = jnp.zeros((_NACC // _NS, MSG), _f32)
    rcv2 = receivers.reshape(E, 1)
    x = nodes
    for p in params["layers"]:
        xs, xr = gather(x, senders, receivers)
        pe, eb = _edge_call(edges, xs, xr, rcv2, p)
        pnum, pden = scatter(pe, eb, receivers, zrows)
        x = _node_call(x, pnum, pden, p)
    cbf = _head_call(x[0:8], params)
    return cbf[0, 0]
